# baseline (speedup 1.0000x reference)
import jax
import jax.numpy as jnp
from jax.experimental import pallas as pl
from jax.experimental.pallas import tpu as pltpu
from jax.experimental.pallas import tpu_sc as plsc
import functools

B, T, D = 32, 4096, 80
BINS = 24
PADN = 4112
IMAX = 2**31 - 1


def _energy_kernel(x_ref, o_ref):
    o_ref[...] = jnp.sum(x_ref[...], axis=-1) / jnp.float32(D)


def _shift_right(x, k):
    z = jnp.zeros((x.shape[0], k), x.dtype)
    return jnp.concatenate([z, x[:, :-k]], axis=1)


def _inblock_scan16(x):
    lane = jax.lax.broadcasted_iota(jnp.int32, x.shape, 1) & 15
    acc = x
    for j in range(1, 16):
        acc = acc + jnp.where(lane == j, _shift_right(acc, 1), jnp.float32(0.0))
    return acc


def _dot(a, b):
    return jnp.dot(a, b, precision=jax.lax.Precision.HIGHEST,
                   preferred_element_type=jnp.float32)


def _emulated_cumsum_4112(cin, sel_mats):
    s1, e1m, s2, e2m = sel_mats
    R = cin.shape[0]
    L1 = _inblock_scan16(cin)
    ends1 = _dot(L1, s1)
    e1p = jnp.concatenate(
        [ends1, jnp.zeros((R, 272 - 257), jnp.float32)], axis=1)
    L2 = _inblock_scan16(e1p)
    ends2 = _dot(L2, s2)
    e2p = jnp.concatenate(
        [ends2, jnp.zeros((R, 32 - 17), jnp.float32)], axis=1)
    L3 = _inblock_scan16(e2p)
    off3 = L3[:, 15:16]
    lane32 = jax.lax.broadcasted_iota(jnp.int32, (R, 32), 1)
    off3_full = jnp.where(lane32 < 16, jnp.float32(0.0),
                          jnp.broadcast_to(off3, (R, 32)))
    F3 = L3 + off3_full
    off2 = jnp.concatenate(
        [jnp.zeros((R, 1), jnp.float32), F3[:, :16]], axis=1)
    F2 = L2 + _dot(off2, e2m)
    off1 = jnp.concatenate(
        [jnp.zeros((R, 1), jnp.float32), F2[:, :256]], axis=1)
    F1 = L1 + _dot(off1, e1m)
    return F1


def _main_kernel(energy_ref, uniform_ref, tp_ref,
                 left_o, right_o, prog_o, pf_o, lr_o, bev_o, sdb_o, voi_o,
                 stats_o):
    f32 = jnp.float32
    energy = energy_ref[...]
    uniform = uniform_ref[...]
    tp = tp_ref[...]

    def rsum(x):
        return jnp.sum(x, axis=1, keepdims=True)

    em = rsum(energy) / f32(T)
    cen = energy - em
    var = rsum(cen * cen) / f32(T - 1)
    es = jnp.maximum(jnp.sqrt(var), f32(1e-6))
    ez = (energy - em) / es

    dif = jnp.abs(energy[:, 1:] - energy[:, :-1])
    delta = jnp.concatenate([jnp.zeros((B, 1), f32), dif], axis=1)

    it_s1 = jax.lax.broadcasted_iota(jnp.int32, (PADN, 257), 0)
    ib_s1 = jax.lax.broadcasted_iota(jnp.int32, (PADN, 257), 1)
    s1 = (it_s1 == 16 * ib_s1 + 15).astype(f32)
    ib_e1 = jax.lax.broadcasted_iota(jnp.int32, (257, PADN), 0)
    it_e1 = jax.lax.broadcasted_iota(jnp.int32, (257, PADN), 1)
    e1m = ((it_e1 >> 4) == ib_e1).astype(f32)
    it_s2 = jax.lax.broadcasted_iota(jnp.int32, (272, 17), 0)
    ib_s2 = jax.lax.broadcasted_iota(jnp.int32, (272, 17), 1)
    s2 = (it_s2 == 16 * ib_s2 + 15).astype(f32)
    ib_e2 = jax.lax.broadcasted_iota(jnp.int32, (17, 272), 0)
    it_e2 = jax.lax.broadcasted_iota(jnp.int32, (17, 272), 1)
    e2m = ((it_e2 >> 4) == ib_e2).astype(f32)
    sel = (s1, e1m, s2, e2m)

    cin5 = jnp.concatenate(
        [jnp.zeros((B, 3), f32), delta, jnp.zeros((B, PADN - T - 3), f32)], axis=1)
    cin7 = jnp.concatenate(
        [jnp.zeros((B, 4), f32), delta, jnp.zeros((B, PADN - T - 4), f32)], axis=1)
    c_all = _emulated_cumsum_4112(
        jnp.concatenate([cin5, cin7], axis=0), sel)
    c5 = c_all[:B]
    c7 = c_all[B:]
    local_rate = (c5[:, 5:4101] - c5[:, :4096]) / f32(5)
    bs = (c7[:, 7:4103] - c7[:, :4096]) / f32(7)

    dbits = jax.lax.bitcast_convert_type(delta, jnp.int32)
    bbits = jax.lax.bitcast_convert_type(bs, jnp.int32)
    bits2 = jnp.concatenate([dbits, bbits], axis=0)
    row2 = jax.lax.broadcasted_iota(jnp.int32, (2 * B, 1), 0)
    kp1 = jnp.where(row2 < B, f32(1434.0), f32(3072.0))

    def body(_, lohi):
        lo, hi = lohi
        mid = lo + (hi - lo) // 2
        cnt = rsum((bits2 <= mid).astype(f32))
        take = cnt >= kp1
        return jnp.where(take, lo, mid + 1), jnp.where(take, mid, hi)

    lo = jnp.zeros((2 * B, 1), jnp.int32)
    hi = jnp.full((2 * B, 1), IMAX)
    lo, hi = jax.lax.fori_loop(0, 31, body, (lo, hi))
    cnt_le = rsum((bits2 <= lo).astype(f32))
    nxt = jnp.min(jnp.where(bits2 > lo, bits2, IMAX), axis=1, keepdims=True)
    hi_bits = jnp.where(cnt_le >= kp1 + f32(1.0), lo, nxt)
    s_lo = jax.lax.bitcast_convert_type(lo, f32)
    s_hi = jax.lax.bitcast_convert_type(hi_bits, f32)
    thr = s_lo * f32(0.75) + s_hi * f32(0.25)
    dthr = thr[:B]
    bthr = thr[B:]

    pause = (ez <= f32(-0.5)) & (delta <= dthr)
    voiced = (ez > f32(-0.1)).astype(f32)
    bev = (bs >= bthr).astype(f32)
    pause_f = pause.astype(f32)

    sp = f32(1.0) - pause_f
    k = 1
    while k < T:
        sp = sp + _shift_right(sp, k)
        k *= 2
    total = jnp.maximum(sp[:, T - 1:T], f32(1.0))
    progress = sp / total
    sdb = progress - uniform

    rights = []
    for j in range(BINS):
        cnt = rsum((progress < tp[:, j:j + 1]).astype(f32))
        rights.append(cnt.astype(jnp.int32))
    right = jnp.concatenate(rights, axis=1)
    left = jnp.clip(right - 1, 0, T - 1)
    r = jnp.clip(right, 0, T - 1)

    left_o[...] = left
    right_o[...] = right
    prog_o[...] = progress
    pf_o[...] = pause_f
    lr_o[...] = local_rate
    bev_o[...] = bev
    sdb_o[...] = sdb
    voi_o[...] = voiced

    half = T // 2
    rate_trend = (rsum(local_rate[:, half:]) / f32(half)
                  - rsum(local_rate[:, :half]) / f32(half))

    def run_mean(mask_f):
        prev = _shift_right(mask_f, 1)
        starts = rsum(jnp.where((mask_f > f32(0.5)) & (prev < f32(0.5)),
                                f32(1.0), f32(0.0)))
        tot = rsum(mask_f)
        return tot / jnp.maximum(starts, f32(1.0))

    speech_f = f32(1.0) - pause_f
    stats_o[...] = jnp.concatenate([
        rsum(pause_f) / f32(T),
        run_mean(pause_f),
        run_mean(speech_f),
        rate_trend,
        rsum(bev) / f32(T),
        rsum(voiced) / f32(T),
    ], axis=1)


NSEG = 22
NREC = B * BINS
PW = NREC // 32


def _sc_resample_kernel(tab_hbm, idx_hbm, right_hbm, tp_hbm, out_hbm,
                        idx_v, gb, rv, tpv, ob, sem):
    f32 = jnp.float32
    wid = jax.lax.axis_index("s") * 2 + jax.lax.axis_index("c")
    base = PW * wid
    for k in range(NSEG):
        pltpu.sync_copy(idx_hbm.at[pl.ds(k * NREC + base, PW)], idx_v.at[k])
    copies = [pltpu.async_copy(tab_hbm.at[idx_v.at[k]], gb.at[k], sem)
              for k in range(NSEG)]
    for c in copies:
        c.wait()
    pltpu.sync_copy(right_hbm.at[pl.ds(base, PW)], rv)
    pltpu.sync_copy(tp_hbm.at[pl.ds(base, PW)], tpv)
    for c in (0, PW - 16):
        ds = pl.ds(c, 16)
        lp = gb[0, ds]
        rp = gb[1, ds]
        tpc = tpv[ds]
        rc = rv[ds]
        denom = jnp.maximum(jnp.abs(rp - lp), f32(1e-6))
        alpha = jnp.clip((tpc - lp) / denom, f32(0.0), f32(1.0))
        lo_e = rc <= 0
        hi_e = rc >= T
        for q in range(5):
            s0 = 2 + 4 * q
            val = gb[s0, ds] * (f32(1.0) - alpha) + gb[s0 + 1, ds] * alpha
            val = jnp.where(lo_e, gb[s0 + 2, ds], val)
            val = jnp.where(hi_e, gb[s0 + 3, ds], val)
            ob[q, ds] = val
    for q in range(5):
        pltpu.sync_copy(ob.at[q], out_hbm.at[pl.ds(q * NREC + base, PW)])


def kernel(ref_mel):
    ref_mel = ref_mel.astype(jnp.float32)
    energy = pl.pallas_call(
        _energy_kernel,
        grid=(4,),
        in_specs=[pl.BlockSpec((8, T, D), lambda i: (i, 0, 0))],
        out_specs=pl.BlockSpec((8, T), lambda i: (i, 0)),
        out_shape=jax.ShapeDtypeStruct((B, T), jnp.float32),
    )(ref_mel)

    uniform = jnp.linspace(0.0, 1.0, T)[None, :]
    tp = jnp.linspace(0.0, 1.0, BINS)[None, :]

    i32 = jnp.int32
    shapes = [jax.ShapeDtypeStruct((B, BINS), i32),
              jax.ShapeDtypeStruct((B, BINS), i32)]
    shapes += [jax.ShapeDtypeStruct((B, T), jnp.float32) for _ in range(6)]
    shapes.append(jax.ShapeDtypeStruct((B, 6), jnp.float32))
    (left, right, progress, pause_f, local_rate, bev, sdb, voiced,
     stats) = pl.pallas_call(
        _main_kernel,
        out_shape=tuple(shapes),
    )(energy, uniform, tp)

    tab = jnp.concatenate([a.reshape(-1) for a in
                           (progress, pause_f, local_rate, bev, sdb, voiced)])
    boff = (jnp.arange(B, dtype=i32) * T)[:, None]
    fl = (boff + left).reshape(-1)
    fr = (boff + jnp.clip(right, 0, T - 1)).reshape(-1)
    ff = jnp.broadcast_to(boff, (B, BINS)).reshape(-1)
    fz = ff + i32(T - 1)
    segs = [fl, fr]
    for q in range(5):
        o = i32((q + 1) * B * T)
        segs += [o + fl, o + fr, o + ff, o + fz]
    idxs = jnp.concatenate(segs)
    right_flat = right.reshape(-1)
    tp_flat = jnp.tile(tp[0], B)

    mesh = plsc.VectorSubcoreMesh(core_axis_name="c", subcore_axis_name="s")
    sck = functools.partial(
        pl.kernel,
        out_type=jax.ShapeDtypeStruct((5 * NREC,), jnp.float32),
        mesh=mesh,
        scratch_types=[
            pltpu.VMEM((NSEG, PW), i32),
            pltpu.VMEM((NSEG, PW), jnp.float32),
            pltpu.VMEM((PW,), i32),
            pltpu.VMEM((PW,), jnp.float32),
            pltpu.VMEM((5, PW), jnp.float32),
            pltpu.SemaphoreType.DMA,
        ],
    )(_sc_resample_kernel)
    vals = sck(tab, idxs, right_flat, tp_flat)

    trace = jnp.moveaxis(vals.reshape(5, B, BINS), 0, -1)
    return trace, stats


# speedup vs baseline: 1.0779x; 1.0779x over previous
import jax
import jax.numpy as jnp
from jax.experimental import pallas as pl
from jax.experimental.pallas import tpu as pltpu
from jax.experimental.pallas import tpu_sc as plsc
import functools

B, T, D = 32, 4096, 80
BINS = 24
PADN = 4112
IMAX = 2**31 - 1


def _energy_kernel(x_ref, o_ref):
    o_ref[...] = jnp.sum(x_ref[...], axis=-1) / jnp.float32(D)


def _shift_right(x, k):
    z = jnp.zeros((x.shape[0], k), x.dtype)
    return jnp.concatenate([z, x[:, :-k]], axis=1)


def _inblock_scan16(x):
    lane = jax.lax.broadcasted_iota(jnp.int32, x.shape, 1) & 15
    acc = x
    for j in range(1, 16):
        acc = acc + jnp.where(lane == j, _shift_right(acc, 1), jnp.float32(0.0))
    return acc


def _dot(a, b):
    return jnp.dot(a, b, precision=jax.lax.Precision.HIGHEST,
                   preferred_element_type=jnp.float32)


def _emulated_cumsum_4112(cin, sel_mats):
    s1, e1m, s2, e2m = sel_mats
    R = cin.shape[0]
    L1 = _inblock_scan16(cin)
    ends1 = _dot(L1, s1)
    e1p = jnp.concatenate(
        [ends1, jnp.zeros((R, 272 - 257), jnp.float32)], axis=1)
    L2 = _inblock_scan16(e1p)
    ends2 = _dot(L2, s2)
    e2p = jnp.concatenate(
        [ends2, jnp.zeros((R, 32 - 17), jnp.float32)], axis=1)
    L3 = _inblock_scan16(e2p)
    off3 = L3[:, 15:16]
    lane32 = jax.lax.broadcasted_iota(jnp.int32, (R, 32), 1)
    off3_full = jnp.where(lane32 < 16, jnp.float32(0.0),
                          jnp.broadcast_to(off3, (R, 32)))
    F3 = L3 + off3_full
    off2 = jnp.concatenate(
        [jnp.zeros((R, 1), jnp.float32), F3[:, :16]], axis=1)
    F2 = L2 + _dot(off2, e2m)
    off1 = jnp.concatenate(
        [jnp.zeros((R, 1), jnp.float32), F2[:, :256]], axis=1)
    F1 = L1 + _dot(off1, e1m)
    return F1


def _main_kernel(energy_ref, uniform_ref, tp_ref,
                 left_o, right_o, tab_o, stats_o):
    f32 = jnp.float32
    energy = energy_ref[...]
    uniform = uniform_ref[...]
    tp = tp_ref[...]

    def rsum(x):
        return jnp.sum(x, axis=1, keepdims=True)

    em = rsum(energy) / f32(T)
    cen = energy - em
    var = rsum(cen * cen) / f32(T - 1)
    es = jnp.maximum(jnp.sqrt(var), f32(1e-6))
    ez = (energy - em) / es

    dif = jnp.abs(energy[:, 1:] - energy[:, :-1])
    delta = jnp.concatenate([jnp.zeros((B, 1), f32), dif], axis=1)

    it_s1 = jax.lax.broadcasted_iota(jnp.int32, (PADN, 257), 0)
    ib_s1 = jax.lax.broadcasted_iota(jnp.int32, (PADN, 257), 1)
    s1 = (it_s1 == 16 * ib_s1 + 15).astype(f32)
    ib_e1 = jax.lax.broadcasted_iota(jnp.int32, (257, PADN), 0)
    it_e1 = jax.lax.broadcasted_iota(jnp.int32, (257, PADN), 1)
    e1m = ((it_e1 >> 4) == ib_e1).astype(f32)
    it_s2 = jax.lax.broadcasted_iota(jnp.int32, (272, 17), 0)
    ib_s2 = jax.lax.broadcasted_iota(jnp.int32, (272, 17), 1)
    s2 = (it_s2 == 16 * ib_s2 + 15).astype(f32)
    ib_e2 = jax.lax.broadcasted_iota(jnp.int32, (17, 272), 0)
    it_e2 = jax.lax.broadcasted_iota(jnp.int32, (17, 272), 1)
    e2m = ((it_e2 >> 4) == ib_e2).astype(f32)
    sel = (s1, e1m, s2, e2m)

    cin5 = jnp.concatenate(
        [jnp.zeros((B, 3), f32), delta, jnp.zeros((B, PADN - T - 3), f32)], axis=1)
    cin7 = jnp.concatenate(
        [jnp.zeros((B, 4), f32), delta, jnp.zeros((B, PADN - T - 4), f32)], axis=1)
    c_all = _emulated_cumsum_4112(
        jnp.concatenate([cin5, cin7], axis=0), sel)
    c5 = c_all[:B]
    c7 = c_all[B:]
    local_rate = (c5[:, 5:4101] - c5[:, :4096]) / f32(5)
    bs = (c7[:, 7:4103] - c7[:, :4096]) / f32(7)

    dbits = jax.lax.bitcast_convert_type(delta, jnp.int32)
    bbits = jax.lax.bitcast_convert_type(bs, jnp.int32)
    bits2 = jnp.concatenate([dbits, bbits], axis=0)
    row2 = jax.lax.broadcasted_iota(jnp.int32, (2 * B, 1), 0)
    kp1 = jnp.where(row2 < B, f32(1434.0), f32(3072.0))

    def body(_, lohi):
        lo, hi = lohi
        mid = lo + (hi - lo) // 2
        cnt = rsum((bits2 <= mid).astype(f32))
        take = cnt >= kp1
        return jnp.where(take, lo, mid + 1), jnp.where(take, mid, hi)

    lo = jnp.zeros((2 * B, 1), jnp.int32)
    hi = jnp.full((2 * B, 1), IMAX)
    lo, hi = jax.lax.fori_loop(0, 31, body, (lo, hi))
    cnt_le = rsum((bits2 <= lo).astype(f32))
    nxt = jnp.min(jnp.where(bits2 > lo, bits2, IMAX), axis=1, keepdims=True)
    hi_bits = jnp.where(cnt_le >= kp1 + f32(1.0), lo, nxt)
    s_lo = jax.lax.bitcast_convert_type(lo, f32)
    s_hi = jax.lax.bitcast_convert_type(hi_bits, f32)
    thr = s_lo * f32(0.75) + s_hi * f32(0.25)
    dthr = thr[:B]
    bthr = thr[B:]

    pause = (ez <= f32(-0.5)) & (delta <= dthr)
    voiced = (ez > f32(-0.1)).astype(f32)
    bev = (bs >= bthr).astype(f32)
    pause_f = pause.astype(f32)

    sp = f32(1.0) - pause_f
    k = 1
    while k < T:
        sp = sp + _shift_right(sp, k)
        k *= 2
    total = jnp.maximum(sp[:, T - 1:T], f32(1.0))
    progress = sp / total
    sdb = progress - uniform

    rights = []
    for j in range(BINS):
        cnt = rsum((progress < tp[:, j:j + 1]).astype(f32))
        rights.append(cnt.astype(jnp.int32))
    right = jnp.concatenate(rights, axis=1)
    left = jnp.clip(right - 1, 0, T - 1)
    r = jnp.clip(right, 0, T - 1)

    left_o[...] = left
    right_o[...] = right
    tab_o[...] = jnp.concatenate(
        [progress, pause_f, local_rate, bev, sdb, voiced], axis=0)

    half = T // 2
    rate_trend = (rsum(local_rate[:, half:]) / f32(half)
                  - rsum(local_rate[:, :half]) / f32(half))

    def run_mean(mask_f):
        prev = _shift_right(mask_f, 1)
        starts = rsum(jnp.where((mask_f > f32(0.5)) & (prev < f32(0.5)),
                                f32(1.0), f32(0.0)))
        tot = rsum(mask_f)
        return tot / jnp.maximum(starts, f32(1.0))

    speech_f = f32(1.0) - pause_f
    stats_o[...] = jnp.concatenate([
        rsum(pause_f) / f32(T),
        run_mean(pause_f),
        run_mean(speech_f),
        rate_trend,
        rsum(bev) / f32(T),
        rsum(voiced) / f32(T),
    ], axis=1)


NSEG = 22
NREC = B * BINS
PW = NREC // 32


def _sc_resample_kernel(tab_hbm, idx_hbm, right_hbm, tp_hbm, out_hbm,
                        idx_v, gb, rv, tpv, ob, sem):
    f32 = jnp.float32
    wid = jax.lax.axis_index("s") * 2 + jax.lax.axis_index("c")
    pltpu.sync_copy(idx_hbm.at[wid], idx_v)
    copies = [pltpu.async_copy(tab_hbm.at[idx_v.at[k]], gb.at[k], sem)
              for k in range(NSEG)]
    pltpu.sync_copy(right_hbm.at[wid], rv)
    pltpu.sync_copy(tp_hbm, tpv)
    for c in copies:
        c.wait()
    for c in (0, PW - 16):
        ds = pl.ds(c, 16)
        lp = gb[0, ds]
        rp = gb[1, ds]
        tpc = tpv[ds]
        rc = rv[ds]
        denom = jnp.maximum(jnp.abs(rp - lp), f32(1e-6))
        alpha = jnp.clip((tpc - lp) / denom, f32(0.0), f32(1.0))
        lo_e = rc <= 0
        hi_e = rc >= T
        for q in range(5):
            s0 = 2 + 4 * q
            val = gb[s0, ds] * (f32(1.0) - alpha) + gb[s0 + 1, ds] * alpha
            val = jnp.where(lo_e, gb[s0 + 2, ds], val)
            val = jnp.where(hi_e, gb[s0 + 3, ds], val)
            ob[q, ds] = val
    pltpu.sync_copy(ob, out_hbm.at[wid])


def kernel(ref_mel):
    ref_mel = ref_mel.astype(jnp.float32)
    energy = pl.pallas_call(
        _energy_kernel,
        grid=(4,),
        in_specs=[pl.BlockSpec((8, T, D), lambda i: (i, 0, 0))],
        out_specs=pl.BlockSpec((8, T), lambda i: (i, 0)),
        out_shape=jax.ShapeDtypeStruct((B, T), jnp.float32),
    )(ref_mel)

    uniform = jnp.linspace(0.0, 1.0, T)[None, :]
    tp = jnp.linspace(0.0, 1.0, BINS)[None, :]

    i32 = jnp.int32
    shapes = [jax.ShapeDtypeStruct((B, BINS), i32),
              jax.ShapeDtypeStruct((B, BINS), i32),
              jax.ShapeDtypeStruct((6 * B, T), jnp.float32),
              jax.ShapeDtypeStruct((B, 6), jnp.float32)]
    left, right, tab, stats = pl.pallas_call(
        _main_kernel,
        out_shape=tuple(shapes),
    )(energy, uniform, tp)

    boff = (jnp.arange(B, dtype=i32) * T)[:, None]
    fl = boff + left
    fr = boff + jnp.clip(right, 0, T - 1)
    ff = jnp.broadcast_to(boff, (B, BINS))
    fz = ff + i32(T - 1)
    segs = [fl, fr]
    for q in range(5):
        o = i32((q + 1) * B * T)
        segs += [o + fl, o + fr, o + ff, o + fz]
    idxs = jnp.stack(segs, axis=1)

    mesh = plsc.VectorSubcoreMesh(core_axis_name="c", subcore_axis_name="s")
    sck = functools.partial(
        pl.kernel,
        out_type=jax.ShapeDtypeStruct((B, 5, BINS), jnp.float32),
        mesh=mesh,
        scratch_types=[
            pltpu.VMEM((NSEG, PW), i32),
            pltpu.VMEM((NSEG, PW), jnp.float32),
            pltpu.VMEM((PW,), i32),
            pltpu.VMEM((PW,), jnp.float32),
            pltpu.VMEM((5, PW), jnp.float32),
            pltpu.SemaphoreType.DMA,
        ],
    )(_sc_resample_kernel)
    vals = sck(tab.reshape(-1), idxs, right, tp[0])

    trace = jnp.transpose(vals, (0, 2, 1))
    return trace, stats


# speedup vs baseline: 1.0909x; 1.0120x over previous
import jax
import jax.numpy as jnp
from jax.experimental import pallas as pl
from jax.experimental.pallas import tpu as pltpu
from jax.experimental.pallas import tpu_sc as plsc
import functools

B, T, D = 32, 4096, 80
BINS = 24
PADN = 4112
IMAX = 2**31 - 1


def _energy_kernel(x_ref, o_ref):
    o_ref[...] = jnp.sum(x_ref[...], axis=-1) / jnp.float32(D)


def _shift_right(x, k):
    z = jnp.zeros((x.shape[0], k), x.dtype)
    return jnp.concatenate([z, x[:, :-k]], axis=1)


def _inblock_scan16(x):
    lane = jax.lax.broadcasted_iota(jnp.int32, x.shape, 1) & 15
    acc = x
    for j in range(1, 16):
        acc = acc + jnp.where(lane == j, _shift_right(acc, 1), jnp.float32(0.0))
    return acc


def _dot(a, b):
    return jnp.dot(a, b, precision=jax.lax.Precision.HIGHEST,
                   preferred_element_type=jnp.float32)


def _emulated_cumsum_4112(cin, sel_mats):
    s1, e1m, s2, e2m = sel_mats
    R = cin.shape[0]
    L1 = _inblock_scan16(cin)
    ends1 = _dot(L1, s1)
    e1p = jnp.concatenate(
        [ends1, jnp.zeros((R, 272 - 257), jnp.float32)], axis=1)
    L2 = _inblock_scan16(e1p)
    ends2 = _dot(L2, s2)
    e2p = jnp.concatenate(
        [ends2, jnp.zeros((R, 32 - 17), jnp.float32)], axis=1)
    L3 = _inblock_scan16(e2p)
    off3 = L3[:, 15:16]
    lane32 = jax.lax.broadcasted_iota(jnp.int32, (R, 32), 1)
    off3_full = jnp.where(lane32 < 16, jnp.float32(0.0),
                          jnp.broadcast_to(off3, (R, 32)))
    F3 = L3 + off3_full
    off2 = jnp.concatenate(
        [jnp.zeros((R, 1), jnp.float32), F3[:, :16]], axis=1)
    F2 = L2 + _dot(off2, e2m)
    off1 = jnp.concatenate(
        [jnp.zeros((R, 1), jnp.float32), F2[:, :256]], axis=1)
    F1 = L1 + _dot(off1, e1m)
    return F1


def _main_kernel(mel_ref, uniform_ref, tp_ref,
                 left_o, right_o, tab_o, stats_o, escr):
    f32 = jnp.float32
    i = pl.program_id(0)

    @pl.when(i < 4)
    def _():
        e = jnp.sum(mel_ref[...], axis=-1) / f32(D)
        escr[pl.ds(8 * i, 8), :] = e

    @pl.when(i == 4)
    def _():
        _phase_b(escr[...], uniform_ref[...], tp_ref[...],
                 left_o, right_o, tab_o, stats_o)


def _phase_b(energy, uniform, tp, left_o, right_o, tab_o, stats_o):
    f32 = jnp.float32
    def rsum(x):
        return jnp.sum(x, axis=1, keepdims=True)

    em = rsum(energy) / f32(T)
    cen = energy - em
    var = rsum(cen * cen) / f32(T - 1)
    es = jnp.maximum(jnp.sqrt(var), f32(1e-6))
    ez = (energy - em) / es

    dif = jnp.abs(energy[:, 1:] - energy[:, :-1])
    delta = jnp.concatenate([jnp.zeros((B, 1), f32), dif], axis=1)

    it_s1 = jax.lax.broadcasted_iota(jnp.int32, (PADN, 257), 0)
    ib_s1 = jax.lax.broadcasted_iota(jnp.int32, (PADN, 257), 1)
    s1 = (it_s1 == 16 * ib_s1 + 15).astype(f32)
    ib_e1 = jax.lax.broadcasted_iota(jnp.int32, (257, PADN), 0)
    it_e1 = jax.lax.broadcasted_iota(jnp.int32, (257, PADN), 1)
    e1m = ((it_e1 >> 4) == ib_e1).astype(f32)
    it_s2 = jax.lax.broadcasted_iota(jnp.int32, (272, 17), 0)
    ib_s2 = jax.lax.broadcasted_iota(jnp.int32, (272, 17), 1)
    s2 = (it_s2 == 16 * ib_s2 + 15).astype(f32)
    ib_e2 = jax.lax.broadcasted_iota(jnp.int32, (17, 272), 0)
    it_e2 = jax.lax.broadcasted_iota(jnp.int32, (17, 272), 1)
    e2m = ((it_e2 >> 4) == ib_e2).astype(f32)
    sel = (s1, e1m, s2, e2m)

    cin5 = jnp.concatenate(
        [jnp.zeros((B, 3), f32), delta, jnp.zeros((B, PADN - T - 3), f32)], axis=1)
    cin7 = jnp.concatenate(
        [jnp.zeros((B, 4), f32), delta, jnp.zeros((B, PADN - T - 4), f32)], axis=1)
    c_all = _emulated_cumsum_4112(
        jnp.concatenate([cin5, cin7], axis=0), sel)
    c5 = c_all[:B]
    c7 = c_all[B:]
    local_rate = (c5[:, 5:4101] - c5[:, :4096]) / f32(5)
    bs = (c7[:, 7:4103] - c7[:, :4096]) / f32(7)

    dbits = jax.lax.bitcast_convert_type(delta, jnp.int32)
    bbits = jax.lax.bitcast_convert_type(bs, jnp.int32)
    bits2 = jnp.concatenate([dbits, bbits], axis=0)
    row2 = jax.lax.broadcasted_iota(jnp.int32, (2 * B, 1), 0)
    kp1 = jnp.where(row2 < B, f32(1434.0), f32(3072.0))

    def body(_, lohi):
        lo, hi = lohi
        mid = lo + (hi - lo) // 2
        cnt = rsum((bits2 <= mid).astype(f32))
        take = cnt >= kp1
        return jnp.where(take, lo, mid + 1), jnp.where(take, mid, hi)

    lo = jnp.zeros((2 * B, 1), jnp.int32)
    hi = jnp.full((2 * B, 1), IMAX)
    lo, hi = jax.lax.fori_loop(0, 31, body, (lo, hi))
    cnt_le = rsum((bits2 <= lo).astype(f32))
    nxt = jnp.min(jnp.where(bits2 > lo, bits2, IMAX), axis=1, keepdims=True)
    hi_bits = jnp.where(cnt_le >= kp1 + f32(1.0), lo, nxt)
    s_lo = jax.lax.bitcast_convert_type(lo, f32)
    s_hi = jax.lax.bitcast_convert_type(hi_bits, f32)
    thr = s_lo * f32(0.75) + s_hi * f32(0.25)
    dthr = thr[:B]
    bthr = thr[B:]

    pause = (ez <= f32(-0.5)) & (delta <= dthr)
    voiced = (ez > f32(-0.1)).astype(f32)
    bev = (bs >= bthr).astype(f32)
    pause_f = pause.astype(f32)

    sp = f32(1.0) - pause_f
    k = 1
    while k < T:
        sp = sp + _shift_right(sp, k)
        k *= 2
    total = jnp.maximum(sp[:, T - 1:T], f32(1.0))
    progress = sp / total
    sdb = progress - uniform

    rights = []
    for j in range(BINS):
        cnt = rsum((progress < tp[:, j:j + 1]).astype(f32))
        rights.append(cnt.astype(jnp.int32))
    right = jnp.concatenate(rights, axis=1)
    left = jnp.clip(right - 1, 0, T - 1)
    r = jnp.clip(right, 0, T - 1)

    left_o[...] = left
    right_o[...] = right
    tab_o[...] = jnp.concatenate(
        [progress, pause_f, local_rate, bev, sdb, voiced], axis=0)

    half = T // 2
    rate_trend = (rsum(local_rate[:, half:]) / f32(half)
                  - rsum(local_rate[:, :half]) / f32(half))

    def run_mean(mask_f):
        prev = _shift_right(mask_f, 1)
        starts = rsum(jnp.where((mask_f > f32(0.5)) & (prev < f32(0.5)),
                                f32(1.0), f32(0.0)))
        tot = rsum(mask_f)
        return tot / jnp.maximum(starts, f32(1.0))

    speech_f = f32(1.0) - pause_f
    stats_o[...] = jnp.concatenate([
        rsum(pause_f) / f32(T),
        run_mean(pause_f),
        run_mean(speech_f),
        rate_trend,
        rsum(bev) / f32(T),
        rsum(voiced) / f32(T),
    ], axis=1)


NSEG = 22
NREC = B * BINS
PW = NREC // 32


def _sc_resample_kernel(tab_hbm, idx_hbm, right_hbm, tp_hbm, out_hbm,
                        idx_v, gb, rv, tpv, ob, sem):
    f32 = jnp.float32
    wid = jax.lax.axis_index("s") * 2 + jax.lax.axis_index("c")
    pltpu.sync_copy(idx_hbm.at[wid], idx_v)
    copies = [pltpu.async_copy(tab_hbm.at[idx_v.at[k]], gb.at[k], sem)
              for k in range(NSEG)]
    pltpu.sync_copy(right_hbm.at[wid], rv)
    pltpu.sync_copy(tp_hbm, tpv)
    for c in copies:
        c.wait()
    for c in (0, PW - 16):
        ds = pl.ds(c, 16)
        lp = gb[0, ds]
        rp = gb[1, ds]
        tpc = tpv[ds]
        rc = rv[ds]
        denom = jnp.maximum(jnp.abs(rp - lp), f32(1e-6))
        alpha = jnp.clip((tpc - lp) / denom, f32(0.0), f32(1.0))
        lo_e = rc <= 0
        hi_e = rc >= T
        for q in range(5):
            s0 = 2 + 4 * q
            val = gb[s0, ds] * (f32(1.0) - alpha) + gb[s0 + 1, ds] * alpha
            val = jnp.where(lo_e, gb[s0 + 2, ds], val)
            val = jnp.where(hi_e, gb[s0 + 3, ds], val)
            ob[q, ds] = val
    pltpu.sync_copy(ob, out_hbm.at[wid])


def kernel(ref_mel):
    ref_mel = ref_mel.astype(jnp.float32)
    uniform = jnp.linspace(0.0, 1.0, T)[None, :]
    tp = jnp.linspace(0.0, 1.0, BINS)[None, :]

    i32 = jnp.int32
    shapes = [jax.ShapeDtypeStruct((B, BINS), i32),
              jax.ShapeDtypeStruct((B, BINS), i32),
              jax.ShapeDtypeStruct((6 * B, T), jnp.float32),
              jax.ShapeDtypeStruct((B, 6), jnp.float32)]
    left, right, tab, stats = pl.pallas_call(
        _main_kernel,
        grid=(5,),
        in_specs=[
            pl.BlockSpec((8, T, D), lambda i: (jnp.minimum(i, 3), 0, 0)),
            pl.BlockSpec((1, T), lambda i: (0, 0)),
            pl.BlockSpec((1, BINS), lambda i: (0, 0)),
        ],
        out_specs=[
            pl.BlockSpec((B, BINS), lambda i: (0, 0)),
            pl.BlockSpec((B, BINS), lambda i: (0, 0)),
            pl.BlockSpec((6 * B, T), lambda i: (0, 0)),
            pl.BlockSpec((B, 6), lambda i: (0, 0)),
        ],
        out_shape=tuple(shapes),
        scratch_shapes=[pltpu.VMEM((B, T), jnp.float32)],
    )(ref_mel, uniform, tp)

    boff = (jnp.arange(B, dtype=i32) * T)[:, None]
    fl = boff + left
    fr = boff + jnp.clip(right, 0, T - 1)
    ff = jnp.broadcast_to(boff, (B, BINS))
    fz = ff + i32(T - 1)
    segs = [fl, fr]
    for q in range(5):
        o = i32((q + 1) * B * T)
        segs += [o + fl, o + fr, o + ff, o + fz]
    idxs = jnp.stack(segs, axis=1)

    mesh = plsc.VectorSubcoreMesh(core_axis_name="c", subcore_axis_name="s")
    sck = functools.partial(
        pl.kernel,
        out_type=jax.ShapeDtypeStruct((B, 5, BINS), jnp.float32),
        mesh=mesh,
        scratch_types=[
            pltpu.VMEM((NSEG, PW), i32),
            pltpu.VMEM((NSEG, PW), jnp.float32),
            pltpu.VMEM((PW,), i32),
            pltpu.VMEM((PW,), jnp.float32),
            pltpu.VMEM((5, PW), jnp.float32),
            pltpu.SemaphoreType.DMA,
        ],
    )(_sc_resample_kernel)
    vals = sck(tab.reshape(-1), idxs, right, tp[0])

    trace = jnp.transpose(vals, (0, 2, 1))
    return trace, stats
